# baseline (speedup 1.0000x reference)
import functools

import jax
import jax.numpy as jnp
from jax import lax
from jax.experimental import pallas as pl
from jax.experimental.pallas import tpu as pltpu
from jax.experimental.pallas import tpu_sc as plsc

N = 10000
NNZ = 160000
D_IN = 256
D_OUT = 256
DH = 128

NC = 2
NS = 16
CH = 64
NCH = 160
EPT = NCH * CH
N2 = 10112
RPT = N2 // NS

PRO = 2
EPI = 8
NBLK = (NCH - PRO - EPI) // 6

_mesh = plsc.VectorSubcoreMesh(
    core_axis_name="c", subcore_axis_name="s", num_cores=NC, num_subcores=NS
)


@functools.partial(
    pl.kernel,
    out_type=jax.ShapeDtypeStruct((NC, N2, DH), jnp.float32),
    mesh=_mesh,
    scratch_types=[
        [pltpu.VMEM((2, CH), jnp.int32) for _ in range(6)],
        [pltpu.VMEM((1, CH), jnp.float32) for _ in range(6)],
        [pltpu.VMEM((CH, DH), jnp.float32) for _ in range(3)],
        pltpu.VMEM_SHARED((N2, DH), jnp.float32),
        [pltpu.SemaphoreType.DMA for _ in range(6)],
        [pltpu.SemaphoreType.DMA for _ in range(3)],
        [pltpu.SemaphoreType.DMA for _ in range(3)],
    ],
)
def _spmm_sc(xs, edata, vdata, out, ibuf, vbuf, gb, acc, isem, gsem, ssem):
    c = lax.axis_index("c")
    s = lax.axis_index("s")

    xh = xs.at[c]
    ed = edata.at[s]
    vd = vdata.at[s]

    def zrow(i, carry):
        zv = jnp.zeros((16,), jnp.float32)
        for j in range(DH // 16):
            gb[0][i, pl.ds(j * 16, 16)] = zv
        return carry

    lax.fori_loop(0, CH, zrow, 0)
    base = s * RPT
    for i in range(RPT // CH):
        pltpu.sync_copy(gb[0], acc.at[pl.ds(base + i * CH, CH)])
    rem = RPT % CH
    pltpu.sync_copy(gb[0].at[pl.ds(0, rem)],
                    acc.at[pl.ds(base + RPT - rem, rem)])
    plsc.subcore_barrier()


    def idx_issue(ci, m):
        pltpu.async_copy(ed.at[ci], ibuf[m], isem[m])
        pltpu.async_copy(vd.at[ci], vbuf[m], isem[m])

    def idx_wait(ci, m):
        pltpu.make_async_copy(ed.at[ci], ibuf[m], isem[m]).wait()
        pltpu.make_async_copy(vd.at[ci], vbuf[m], isem[m]).wait()

    def g_issue(b, m):
        pltpu.async_copy(xh.at[ibuf[m].at[1]], gb[b], gsem[b])

    def g_wait(b, m):
        pltpu.make_async_copy(xh.at[ibuf[m].at[1]], gb[b], gsem[b]).wait()

    def s_issue(b, m):
        pltpu.async_copy(gb[b], acc.at[ibuf[m].at[0]], ssem[b], add=True)

    def s_wait(b, m):
        pltpu.make_async_copy(gb[b], acc.at[ibuf[m].at[0]], ssem[b]).wait()

    def scale(b, m):
        def group_body(g, carry):
            vv = vbuf[m][0, pl.ds(g * 16, 16)]
            for e in range(16):
                v = vv[e]
                row = g * 16 + e
                for j in range(DH // 16):
                    sl = pl.ds(j * 16, 16)
                    gb[b][row, sl] = gb[b][row, sl] * v
            return carry

        lax.fori_loop(0, CH // 16, group_body, 0)

    def slot(ci, m, do_sswait=True, do_gather=True, do_idx=True):
        X = m % 3
        Z = (m + 2) % 3
        g_wait(X, m)
        scale(X, m)
        s_issue(X, m)
        if do_gather:
            idx_wait(ci + 2, (m + 2) % 6)
        if do_sswait:
            s_wait(Z, (m + 5) % 6)
        if do_gather:
            g_issue(Z, (m + 2) % 6)
        if do_idx:
            idx_issue(ci + 4, (m + 4) % 6)

    for k in range(4):
        idx_issue(k, k)
    idx_wait(0, 0)
    g_issue(0, 0)
    idx_wait(1, 1)
    g_issue(1, 1)
    slot(0, 0, do_sswait=False)
    slot(1, 1)

    def block_body(u, carry):
        c0 = PRO + 6 * u
        for d in range(6):
            slot(c0 + d, (PRO + d) % 6)
        return carry

    lax.fori_loop(0, NBLK, block_body, 0)

    for ci in range(NCH - EPI, NCH):
        slot(ci, ci % 6,
             do_gather=(ci + 2 < NCH), do_idx=(ci + 4 < NCH))
    s_wait((NCH - 1) % 3, (NCH - 1) % 6)
    plsc.subcore_barrier()

    osl = pl.ds(base, RPT)
    pltpu.sync_copy(acc.at[osl], out.at[c].at[osl])


_TM = 1000


def _linear_relu_body(hs_ref, wt_ref, o_ref):
    hl = hs_ref[0]
    hr = hs_ref[1]
    acc = jnp.dot(hl, wt_ref[:DH], preferred_element_type=jnp.float32)
    acc += jnp.dot(hr, wt_ref[DH:], preferred_element_type=jnp.float32)
    o_ref[...] = jnp.maximum(acc, 0.0)


_linear_relu = pl.pallas_call(
    _linear_relu_body,
    grid=(N // _TM,),
    in_specs=[
        pl.BlockSpec((NC, _TM, DH), lambda i: (0, i, 0)),
        pl.BlockSpec((D_IN, D_OUT), lambda i: (0, 0)),
    ],
    out_specs=pl.BlockSpec((_TM, D_OUT), lambda i: (i, 0)),
    out_shape=jax.ShapeDtypeStruct((N, D_OUT), jnp.float32),
)


def kernel(x, A_indices, A_values, shape, W):
    del shape
    pad = EPT - NNZ // NS
    rows = jnp.pad(A_indices[0].astype(jnp.int32).reshape(NS, NNZ // NS),
                   ((0, 0), (0, pad)))
    cols = jnp.pad(A_indices[1].astype(jnp.int32).reshape(NS, NNZ // NS),
                   ((0, 0), (0, pad)))
    vals = jnp.pad(A_values.astype(jnp.float32).reshape(NS, NNZ // NS),
                   ((0, 0), (0, pad)))
    edata = jnp.stack(
        [rows.reshape(NS, NCH, CH), cols.reshape(NS, NCH, CH)], axis=2
    )
    vdata = vals.reshape(NS, NCH, 1, CH)
    xs = jnp.stack([x[:, :DH], x[:, DH:]])
    hs = _spmm_sc(xs, edata, vdata)
    return _linear_relu(hs, W.T.astype(jnp.float32))


# speedup vs baseline: 1.0100x; 1.0100x over previous
import functools

import jax
import jax.numpy as jnp
from jax import lax
from jax.experimental import pallas as pl
from jax.experimental.pallas import tpu as pltpu
from jax.experimental.pallas import tpu_sc as plsc

N = 10000
NNZ = 160000
D_IN = 256
D_OUT = 256
DH = 128

NC = 2
NS = 16
CH = 64
NCH = 160
EPT = NCH * CH
CPS = 32
NSC = NCH // CPS
N2 = 10112
RPT = N2 // NS

_mesh = plsc.VectorSubcoreMesh(
    core_axis_name="c", subcore_axis_name="s", num_cores=NC, num_subcores=NS
)


@functools.partial(
    pl.kernel,
    out_type=jax.ShapeDtypeStruct((NC, N2, DH), jnp.float32),
    mesh=_mesh,
    scratch_types=[
        pltpu.VMEM((CPS, CH), jnp.int32),
        pltpu.VMEM((CPS, CH), jnp.int32),
        pltpu.VMEM((CPS, CH), jnp.float32),
        [pltpu.VMEM((CH, DH), jnp.float32) for _ in range(3)],
        pltpu.VMEM_SHARED((N2, DH), jnp.float32),
        [pltpu.SemaphoreType.DMA for _ in range(3)],
        [pltpu.SemaphoreType.DMA for _ in range(3)],
    ],
)
def _spmm_sc(xs, rows, cols, vals, out, row_v, col_v, val_v, gb, acc, gsem, ssem):
    c = lax.axis_index("c")
    s = lax.axis_index("s")

    xh = xs.at[c]

    def zrow(i, carry):
        zv = jnp.zeros((16,), jnp.float32)
        for j in range(DH // 16):
            gb[0][i, pl.ds(j * 16, 16)] = zv
        return carry

    lax.fori_loop(0, CH, zrow, 0)
    base = s * RPT
    for i in range(RPT // CH):
        pltpu.sync_copy(gb[0], acc.at[pl.ds(base + i * CH, CH)])
    rem = RPT % CH
    pltpu.sync_copy(gb[0].at[pl.ds(0, rem)],
                    acc.at[pl.ds(base + RPT - rem, rem)])
    plsc.subcore_barrier()


    def g_issue(b, ci):
        pltpu.async_copy(xh.at[col_v.at[ci]], gb[b], gsem[b])

    def g_wait(b, ci):
        pltpu.make_async_copy(xh.at[col_v.at[ci]], gb[b], gsem[b]).wait()

    def s_issue(b, ci):
        pltpu.async_copy(gb[b], acc.at[row_v.at[ci]], ssem[b], add=True)

    def s_wait(b, ci):
        pltpu.make_async_copy(gb[b], acc.at[row_v.at[ci]], ssem[b]).wait()

    def scale(b, ci):
        def group_body(g, carry):
            vv = val_v[ci, pl.ds(g * 16, 16)]
            for e in range(16):
                v = vv[e]
                row = g * 16 + e
                for j in range(DH // 16):
                    sl = pl.ds(j * 16, 16)
                    gb[b][row, sl] = gb[b][row, sl] * v
            return carry

        lax.fori_loop(0, CH // 16, group_body, 0)

    def slot(ci, m, do_sswait=True, do_gather=True):
        X = m
        Z = (m + 2) % 3
        g_wait(X, ci)
        scale(X, ci)
        s_issue(X, ci)
        if do_sswait:
            s_wait(Z, ci - 1)
        if do_gather:
            g_issue(Z, ci + 2)

    def super_body(sc, carry0):
        pltpu.sync_copy(rows.at[s].at[sc], row_v)
        pltpu.sync_copy(cols.at[s].at[sc], col_v)
        pltpu.sync_copy(vals.at[s].at[sc], val_v)

        g_issue(0, 0)
        g_issue(1, 1)
        slot(0, 0, do_sswait=False)
        slot(1, 1)

        def block_body(u, carry):
            c0 = 2 + 3 * u
            for d in range(3):
                slot(c0 + d, (2 + d) % 3)
            return carry

        lax.fori_loop(0, (CPS - 5) // 3, block_body, 0)

        for ci in range(CPS - 3, CPS):
            slot(ci, ci % 3, do_gather=(ci + 2 < CPS))
        s_wait((CPS - 1) % 3, CPS - 1)
        return carry0

    lax.fori_loop(0, NSC, super_body, 0)
    plsc.subcore_barrier()

    osl = pl.ds(base, RPT)
    pltpu.sync_copy(acc.at[osl], out.at[c].at[osl])


_TM = 1000


def _linear_relu_body(hs_ref, wt_ref, o_ref):
    hl = hs_ref[0]
    hr = hs_ref[1]
    acc = jnp.dot(hl, wt_ref[:DH], preferred_element_type=jnp.float32)
    acc += jnp.dot(hr, wt_ref[DH:], preferred_element_type=jnp.float32)
    o_ref[...] = jnp.maximum(acc, 0.0)


_linear_relu = pl.pallas_call(
    _linear_relu_body,
    grid=(N // _TM,),
    in_specs=[
        pl.BlockSpec((NC, _TM, DH), lambda i: (0, i, 0)),
        pl.BlockSpec((D_IN, D_OUT), lambda i: (0, 0)),
    ],
    out_specs=pl.BlockSpec((_TM, D_OUT), lambda i: (i, 0)),
    out_shape=jax.ShapeDtypeStruct((N, D_OUT), jnp.float32),
)


def kernel(x, A_indices, A_values, shape, W):
    del shape
    pad = EPT - NNZ // NS
    rows = jnp.pad(A_indices[0].astype(jnp.int32).reshape(NS, NNZ // NS),
                   ((0, 0), (0, pad))).reshape(NS, NSC, CPS, CH)
    cols = jnp.pad(A_indices[1].astype(jnp.int32).reshape(NS, NNZ // NS),
                   ((0, 0), (0, pad))).reshape(NS, NSC, CPS, CH)
    vals = jnp.pad(A_values.astype(jnp.float32).reshape(NS, NNZ // NS),
                   ((0, 0), (0, pad))).reshape(NS, NSC, CPS, CH)
    xs = jnp.stack([x[:, :DH], x[:, DH:]])
    hs = _spmm_sc(xs, rows, cols, vals)
    return _linear_relu(hs, W.T.astype(jnp.float32))


# speedup vs baseline: 1.3664x; 1.3528x over previous
import functools

import jax
import jax.numpy as jnp
from jax import lax
from jax.experimental import pallas as pl
from jax.experimental.pallas import tpu as pltpu
from jax.experimental.pallas import tpu_sc as plsc

N = 10000
NNZ = 160000
D_IN = 256
D_OUT = 256
DH = 128

NC = 2
NS = 16
EPT = 10080
CH = 96
NCH = 105
CPS = 21
NSC = NCH // CPS
NPAIR = (CPS - 1) // 2 - 1
N2 = 10112
RPT = N2 // NS

_mesh = plsc.VectorSubcoreMesh(
    core_axis_name="c", subcore_axis_name="s", num_cores=NC, num_subcores=NS
)


@functools.partial(
    pl.kernel,
    out_type=jax.ShapeDtypeStruct((NC, N2, DH), jnp.float32),
    mesh=_mesh,
    scratch_types=[
        pltpu.VMEM((CPS, CH), jnp.int32),
        pltpu.VMEM((CPS, CH), jnp.int32),
        pltpu.VMEM((CPS, CH), jnp.float32),
        pltpu.VMEM((CH, DH), jnp.float32),
        pltpu.VMEM((CH, DH), jnp.float32),
        pltpu.VMEM_SHARED((N2, DH), jnp.float32),
        pltpu.SemaphoreType.DMA,
        pltpu.SemaphoreType.DMA,
        pltpu.SemaphoreType.DMA,
        pltpu.SemaphoreType.DMA,
    ],
)
def _spmm_sc(xs, rows, cols, vals, out,
             row_v, col_v, val_v, gba, gbb, acc, gsa, gsb, ssa, ssb):
    c = lax.axis_index("c")
    s = lax.axis_index("s")

    xh = xs.at[c]

    def zrow(i, carry):
        zv = jnp.zeros((16,), jnp.float32)
        for j in range(DH // 16):
            gba[i, pl.ds(j * 16, 16)] = zv
        return carry

    lax.fori_loop(0, CH, zrow, 0)
    base = s * RPT
    for i in range(RPT // CH):
        pltpu.sync_copy(gba, acc.at[pl.ds(base + i * CH, CH)])
    rem = RPT % CH
    pltpu.sync_copy(gba.at[pl.ds(0, rem)],
                    acc.at[pl.ds(base + RPT - rem, rem)])
    plsc.subcore_barrier()


    def g_issue(ci, buf, sem):
        pltpu.async_copy(xh.at[col_v.at[ci]], buf, sem)

    def g_wait(ci, buf, sem):
        pltpu.make_async_copy(xh.at[col_v.at[ci]], buf, sem).wait()

    def s_issue(ci, buf, sem):
        pltpu.async_copy(buf, acc.at[row_v.at[ci]], sem, add=True)

    def s_wait(ci, buf, sem):
        pltpu.make_async_copy(buf, acc.at[row_v.at[ci]], sem).wait()

    def scale(buf, ci):
        def group_body(g, carry):
            vv = val_v[ci, pl.ds(g * 16, 16)]
            for e in range(16):
                v = vv[e]
                row = g * 16 + e
                for j in range(DH // 16):
                    sl = pl.ds(j * 16, 16)
                    buf[row, sl] = buf[row, sl] * v
            return carry

        lax.fori_loop(0, CH // 16, group_body, 0)

    def super_body(sc, carry0):
        pltpu.sync_copy(rows.at[s].at[sc], row_v)
        pltpu.sync_copy(cols.at[s].at[sc], col_v)
        pltpu.sync_copy(vals.at[s].at[sc], val_v)

        g_issue(0, gba, gsa)
        g_issue(1, gbb, gsb)
        g_wait(0, gba, gsa)
        scale(gba, 0)
        s_issue(0, gba, ssa)

        def pair_body(k, carry):
            c1 = 2 * k + 1
            c2 = 2 * k + 2
            s_wait(c2 - 2, gba, ssa)
            g_issue(c2, gba, gsa)
            g_wait(c1, gbb, gsb)
            scale(gbb, c1)
            s_issue(c1, gbb, ssb)
            s_wait(c1, gbb, ssb)
            g_issue(c1 + 2, gbb, gsb)
            g_wait(c2, gba, gsa)
            scale(gba, c2)
            s_issue(c2, gba, ssa)
            return carry

        lax.fori_loop(0, NPAIR, pair_body, 0)

        s_wait(CPS - 3, gba, ssa)
        g_issue(CPS - 1, gba, gsa)
        g_wait(CPS - 2, gbb, gsb)
        scale(gbb, CPS - 2)
        s_issue(CPS - 2, gbb, ssb)
        g_wait(CPS - 1, gba, gsa)
        scale(gba, CPS - 1)
        s_issue(CPS - 1, gba, ssa)
        s_wait(CPS - 2, gbb, ssb)
        s_wait(CPS - 1, gba, ssa)
        return carry0

    lax.fori_loop(0, NSC, super_body, 0)
    plsc.subcore_barrier()

    osl = pl.ds(base, RPT)
    pltpu.sync_copy(acc.at[osl], out.at[c].at[osl])


_TM = 1000


def _linear_relu_body(hs_ref, wt_ref, o_ref):
    hl = hs_ref[0]
    hr = hs_ref[1]
    acc = jnp.dot(hl, wt_ref[:DH], preferred_element_type=jnp.float32)
    acc += jnp.dot(hr, wt_ref[DH:], preferred_element_type=jnp.float32)
    o_ref[...] = jnp.maximum(acc, 0.0)


_linear_relu = pl.pallas_call(
    _linear_relu_body,
    grid=(N // _TM,),
    in_specs=[
        pl.BlockSpec((NC, _TM, DH), lambda i: (0, i, 0)),
        pl.BlockSpec((D_IN, D_OUT), lambda i: (0, 0)),
    ],
    out_specs=pl.BlockSpec((_TM, D_OUT), lambda i: (i, 0)),
    out_shape=jax.ShapeDtypeStruct((N, D_OUT), jnp.float32),
)


def kernel(x, A_indices, A_values, shape, W):
    del shape
    pad = EPT - NNZ // NS
    rows = jnp.pad(A_indices[0].astype(jnp.int32).reshape(NS, NNZ // NS),
                   ((0, 0), (0, pad))).reshape(NS, NSC, CPS, CH)
    cols = jnp.pad(A_indices[1].astype(jnp.int32).reshape(NS, NNZ // NS),
                   ((0, 0), (0, pad))).reshape(NS, NSC, CPS, CH)
    vals = jnp.pad(A_values.astype(jnp.float32).reshape(NS, NNZ // NS),
                   ((0, 0), (0, pad))).reshape(NS, NSC, CPS, CH)
    xs = jnp.stack([x[:, :DH], x[:, DH:]])
    hs = _spmm_sc(xs, rows, cols, vals)
    return _linear_relu(hs, W.T.astype(jnp.float32))


# speedup vs baseline: 1.7081x; 1.2501x over previous
import functools

import jax
import jax.numpy as jnp
from jax import lax
from jax.experimental import pallas as pl
from jax.experimental.pallas import tpu as pltpu
from jax.experimental.pallas import tpu_sc as plsc

N = 10000
NNZ = 160000
D_IN = 256
D_OUT = 256
DH = 128

NC = 2
NS = 16
EPT = 10080
CH = 96
NCH = 105
CPS = 21
NSC = NCH // CPS
NPAIR = (CPS - 1) // 2 - 1
N2 = 10112
RPT = N2 // NS

_mesh = plsc.VectorSubcoreMesh(
    core_axis_name="c", subcore_axis_name="s", num_cores=NC, num_subcores=NS
)


@functools.partial(
    pl.kernel,
    out_type=jax.ShapeDtypeStruct((NC, N2, DH), jnp.float32),
    mesh=_mesh,
    scratch_types=[
        pltpu.VMEM((CPS, CH), jnp.int32),
        pltpu.VMEM((CPS, CH), jnp.int32),
        pltpu.VMEM((CPS, CH), jnp.float32),
        pltpu.VMEM((CH, DH), jnp.float32),
        pltpu.VMEM((CH, DH), jnp.float32),
        pltpu.VMEM_SHARED((N2, DH), jnp.float32),
        pltpu.SemaphoreType.DMA,
        pltpu.SemaphoreType.DMA,
        pltpu.SemaphoreType.DMA,
        pltpu.SemaphoreType.DMA,
    ],
)
def _spmm_sc(xs, rows, cols, vals, out,
             row_v, col_v, val_v, gba, gbb, acc, gsa, gsb, ssa, ssb):
    c = lax.axis_index("c")
    s = lax.axis_index("s")

    xh = xs.at[c]

    def zrow(i, carry):
        zv = jnp.zeros((16,), jnp.float32)
        for j in range(DH // 16):
            gba[i, pl.ds(j * 16, 16)] = zv
        return carry

    lax.fori_loop(0, CH, zrow, 0)
    base = s * RPT
    for i in range(RPT // CH):
        pltpu.sync_copy(gba, acc.at[pl.ds(base + i * CH, CH)])
    rem = RPT % CH
    pltpu.sync_copy(gba.at[pl.ds(0, rem)],
                    acc.at[pl.ds(base + RPT - rem, rem)])
    plsc.subcore_barrier()


    def g_issue(ci, buf, sem):
        pltpu.async_copy(xh.at[col_v.at[ci]], buf, sem)

    def g_wait(ci, buf, sem):
        pltpu.make_async_copy(xh.at[col_v.at[ci]], buf, sem).wait()

    def s_issue(ci, buf, sem):
        pltpu.async_copy(buf, acc.at[row_v.at[ci]], sem, add=True)

    def s_wait(ci, buf, sem):
        pltpu.make_async_copy(buf, acc.at[row_v.at[ci]], sem).wait()

    def scale(buf, ci):
        def group_body(g, carry):
            vv = val_v[ci, pl.ds(g * 16, 16)]
            for e in range(16):
                v = vv[e]
                row = g * 16 + e
                for j in range(DH // 16):
                    sl = pl.ds(j * 16, 16)
                    buf[row, sl] = buf[row, sl] * v
            return carry

        lax.fori_loop(0, CH // 16, group_body, 0)

    def super_body(sc, carry0):
        pltpu.sync_copy(rows.at[s].at[sc], row_v)
        pltpu.sync_copy(cols.at[s].at[sc], col_v)
        pltpu.sync_copy(vals.at[s].at[sc], val_v)

        g_issue(0, gba, gsa)
        g_issue(1, gbb, gsb)
        g_wait(0, gba, gsa)
        scale(gba, 0)
        s_issue(0, gba, ssa)

        def pair_body(k, carry):
            c1 = 2 * k + 1
            c2 = 2 * k + 2
            s_wait(c2 - 2, gba, ssa)
            g_issue(c2, gba, gsa)
            g_wait(c1, gbb, gsb)
            scale(gbb, c1)
            s_issue(c1, gbb, ssb)
            s_wait(c1, gbb, ssb)
            g_issue(c1 + 2, gbb, gsb)
            g_wait(c2, gba, gsa)
            scale(gba, c2)
            s_issue(c2, gba, ssa)
            return carry

        lax.fori_loop(0, NPAIR, pair_body, 0)

        s_wait(CPS - 3, gba, ssa)
        g_issue(CPS - 1, gba, gsa)
        g_wait(CPS - 2, gbb, gsb)
        scale(gbb, CPS - 2)
        s_issue(CPS - 2, gbb, ssb)
        g_wait(CPS - 1, gba, gsa)
        scale(gba, CPS - 1)
        s_issue(CPS - 1, gba, ssa)
        s_wait(CPS - 2, gbb, ssb)
        s_wait(CPS - 1, gba, ssa)
        return carry0

    lax.fori_loop(0, NSC, super_body, 0)
    plsc.subcore_barrier()

    osl = pl.ds(base, RPT)
    pltpu.sync_copy(acc.at[osl], out.at[c].at[osl])


_TM = 1000


def _linear_relu_body(hs_ref, wt_ref, o_ref):
    hl = hs_ref[0]
    hr = hs_ref[1]
    acc = jnp.dot(hl, wt_ref[:DH], preferred_element_type=jnp.float32)
    acc += jnp.dot(hr, wt_ref[DH:], preferred_element_type=jnp.float32)
    o_ref[...] = jnp.maximum(acc, 0.0)


_linear_relu = pl.pallas_call(
    _linear_relu_body,
    grid=(N // _TM,),
    in_specs=[
        pl.BlockSpec((NC, _TM, DH), lambda i: (0, i, 0)),
        pl.BlockSpec((D_IN, D_OUT), lambda i: (0, 0)),
    ],
    out_specs=pl.BlockSpec((_TM, D_OUT), lambda i: (i, 0)),
    out_shape=jax.ShapeDtypeStruct((N, D_OUT), jnp.float32),
)


def kernel(x, A_indices, A_values, shape, W):
    del shape
    pad = EPT - NNZ // NS
    spread = (jnp.arange(pad, dtype=jnp.int32)[None, :] * NS
              + jnp.arange(NS, dtype=jnp.int32)[:, None])
    rows = jnp.concatenate(
        [A_indices[0].astype(jnp.int32).reshape(NS, NNZ // NS), spread % N2],
        axis=1).reshape(NS, NSC, CPS, CH)
    cols = jnp.concatenate(
        [A_indices[1].astype(jnp.int32).reshape(NS, NNZ // NS), spread % N],
        axis=1).reshape(NS, NSC, CPS, CH)
    vals = jnp.pad(A_values.astype(jnp.float32).reshape(NS, NNZ // NS),
                   ((0, 0), (0, pad))).reshape(NS, NSC, CPS, CH)
    xs = jnp.stack([x[:, :DH], x[:, DH:]])
    hs = _spmm_sc(xs, rows, cols, vals)
    return _linear_relu(hs, W.T.astype(jnp.float32))


# speedup vs baseline: 1.7321x; 1.0141x over previous
import functools

import jax
import jax.numpy as jnp
from jax import lax
from jax.experimental import pallas as pl
from jax.experimental.pallas import tpu as pltpu
from jax.experimental.pallas import tpu_sc as plsc

N = 10000
NNZ = 160000
D_IN = 256
D_OUT = 256
DH = 128

NC = 2
NS = 16
EPT = 10240
CH = 64
NCH = 160
CPS = 32
NSC = NCH // CPS
N2 = 10112
RPT = N2 // NS

_mesh = plsc.VectorSubcoreMesh(
    core_axis_name="c", subcore_axis_name="s", num_cores=NC, num_subcores=NS
)


@functools.partial(
    pl.kernel,
    out_type=jax.ShapeDtypeStruct((NC, N2, DH), jnp.float32),
    mesh=_mesh,
    scratch_types=[
        pltpu.VMEM((CPS, CH), jnp.int32),
        pltpu.VMEM((CPS, CH), jnp.int32),
        pltpu.VMEM((CPS, CH), jnp.float32),
        [pltpu.VMEM((CH, DH), jnp.float32) for _ in range(3)],
        pltpu.VMEM_SHARED((N2, DH), jnp.float32),
        [pltpu.SemaphoreType.DMA for _ in range(3)],
        [pltpu.SemaphoreType.DMA for _ in range(3)],
    ],
)
def _spmm_sc(xs, rows, cols, vals, out,
             row_v, col_v, val_v, gb, acc, gsem, ssem):
    c = lax.axis_index("c")
    s = lax.axis_index("s")

    xh = xs.at[c]

    def zrow(i, carry):
        zv = jnp.zeros((16,), jnp.float32)
        for j in range(DH // 16):
            gb[0][i, pl.ds(j * 16, 16)] = zv
        return carry

    lax.fori_loop(0, CH, zrow, 0)
    base = s * RPT
    for i in range(RPT // CH):
        pltpu.sync_copy(gb[0], acc.at[pl.ds(base + i * CH, CH)])
    rem = RPT % CH
    pltpu.sync_copy(gb[0].at[pl.ds(0, rem)],
                    acc.at[pl.ds(base + RPT - rem, rem)])
    plsc.subcore_barrier()


    def g_issue(b, ci):
        pltpu.async_copy(xh.at[col_v.at[ci]], gb[b], gsem[b])

    def g_wait(b, ci):
        pltpu.make_async_copy(xh.at[col_v.at[ci]], gb[b], gsem[b]).wait()

    def s_issue(b, ci):
        pltpu.async_copy(gb[b], acc.at[row_v.at[ci]], ssem[b], add=True)

    def s_wait(b, ci):
        pltpu.make_async_copy(gb[b], acc.at[row_v.at[ci]], ssem[b]).wait()

    def scale(b, ci):
        def group_body(g, carry):
            vv = val_v[ci, pl.ds(g * 16, 16)]
            for e in range(16):
                v = vv[e]
                row = g * 16 + e
                for j in range(DH // 16):
                    sl = pl.ds(j * 16, 16)
                    gb[b][row, sl] = gb[b][row, sl] * v
            return carry

        lax.fori_loop(0, CH // 16, group_body, 0)

    def slot(ci, m, do_sswait=True, do_gather=True):
        X = m
        Z = (m + 2) % 3
        g_wait(X, ci)
        scale(X, ci)
        s_issue(X, ci)
        if do_sswait:
            s_wait(Z, ci - 1)
        if do_gather:
            g_issue(Z, ci + 2)

    def super_body(sc, carry0):
        pltpu.sync_copy(rows.at[s].at[sc], row_v)
        pltpu.sync_copy(cols.at[s].at[sc], col_v)
        pltpu.sync_copy(vals.at[s].at[sc], val_v)

        g_issue(0, 0)
        g_issue(1, 1)
        slot(0, 0, do_sswait=False)
        slot(1, 1)

        def block_body(u, carry):
            c0 = 2 + 3 * u
            for d in range(3):
                slot(c0 + d, (2 + d) % 3)
            return carry

        lax.fori_loop(0, (CPS - 5) // 3, block_body, 0)

        for ci in range(CPS - 3, CPS):
            slot(ci, ci % 3, do_gather=(ci + 2 < CPS))
        s_wait((CPS - 1) % 3, CPS - 1)
        return carry0

    lax.fori_loop(0, NSC, super_body, 0)
    plsc.subcore_barrier()

    osl = pl.ds(base, RPT)
    pltpu.sync_copy(acc.at[osl], out.at[c].at[osl])


_TM = 1000


def _linear_relu_body(hs_ref, wt_ref, o_ref):
    hl = hs_ref[0]
    hr = hs_ref[1]
    acc = jnp.dot(hl, wt_ref[:DH], preferred_element_type=jnp.float32)
    acc += jnp.dot(hr, wt_ref[DH:], preferred_element_type=jnp.float32)
    o_ref[...] = jnp.maximum(acc, 0.0)


_linear_relu = pl.pallas_call(
    _linear_relu_body,
    grid=(N // _TM,),
    in_specs=[
        pl.BlockSpec((NC, _TM, DH), lambda i: (0, i, 0)),
        pl.BlockSpec((D_IN, D_OUT), lambda i: (0, 0)),
    ],
    out_specs=pl.BlockSpec((_TM, D_OUT), lambda i: (i, 0)),
    out_shape=jax.ShapeDtypeStruct((N, D_OUT), jnp.float32),
)


def kernel(x, A_indices, A_values, shape, W):
    del shape
    pad = EPT - NNZ // NS
    spread = (jnp.arange(pad, dtype=jnp.int32)[None, :] * NS
              + jnp.arange(NS, dtype=jnp.int32)[:, None])
    rows = jnp.concatenate(
        [A_indices[0].astype(jnp.int32).reshape(NS, NNZ // NS), spread % N2],
        axis=1).reshape(NS, NSC, CPS, CH)
    cols = jnp.concatenate(
        [A_indices[1].astype(jnp.int32).reshape(NS, NNZ // NS), spread % N],
        axis=1).reshape(NS, NSC, CPS, CH)
    vals = jnp.pad(A_values.astype(jnp.float32).reshape(NS, NNZ // NS),
                   ((0, 0), (0, pad))).reshape(NS, NSC, CPS, CH)
    xs = jnp.stack([x[:, :DH], x[:, DH:]])
    hs = _spmm_sc(xs, rows, cols, vals)
    return _linear_relu(hs, W.T.astype(jnp.float32))


# speedup vs baseline: 1.8444x; 1.0648x over previous
import functools

import jax
import jax.numpy as jnp
from jax import lax
from jax.experimental import pallas as pl
from jax.experimental.pallas import tpu as pltpu
from jax.experimental.pallas import tpu_sc as plsc

N = 10000
NNZ = 160000
D_IN = 256
D_OUT = 256
DH = 128

NC = 2
NS = 16
EPT = 10240
CH = 64
NCH = 160
CPS = 32
NSC = NCH // CPS
N2 = 10112
RPT = N2 // NS

_mesh = plsc.VectorSubcoreMesh(
    core_axis_name="c", subcore_axis_name="s", num_cores=NC, num_subcores=NS
)


@functools.partial(
    pl.kernel,
    out_type=jax.ShapeDtypeStruct((NC, N2, DH), jnp.float32),
    mesh=_mesh,
    scratch_types=[
        pltpu.VMEM((CPS, CH), jnp.int32),
        pltpu.VMEM((CPS, CH), jnp.int32),
        pltpu.VMEM((CPS, CH), jnp.float32),
        [pltpu.VMEM((CH, DH), jnp.float32) for _ in range(3)],
        pltpu.VMEM_SHARED((N2, DH), jnp.float32),
        [pltpu.SemaphoreType.DMA for _ in range(3)],
        [pltpu.SemaphoreType.DMA for _ in range(3)],
    ],
)
def _spmm_sc(x, rows, cols, vals, out,
             row_v, col_v, val_v, gb, acc, gsem, ssem):
    c = lax.axis_index("c")
    s = lax.axis_index("s")

    xh = x.at[:, pl.ds(c * DH, DH)]

    def zrow(i, carry):
        zv = jnp.zeros((16,), jnp.float32)
        for j in range(DH // 16):
            gb[0][i, pl.ds(j * 16, 16)] = zv
        return carry

    lax.fori_loop(0, CH, zrow, 0)
    base = s * RPT
    for i in range(RPT // CH):
        pltpu.sync_copy(gb[0], acc.at[pl.ds(base + i * CH, CH)])
    rem = RPT % CH
    pltpu.sync_copy(gb[0].at[pl.ds(0, rem)],
                    acc.at[pl.ds(base + RPT - rem, rem)])
    plsc.subcore_barrier()


    def g_issue(b, ci):
        pltpu.async_copy(xh.at[col_v.at[ci]], gb[b], gsem[b])

    def g_wait(b, ci):
        pltpu.make_async_copy(xh.at[col_v.at[ci]], gb[b], gsem[b]).wait()

    def s_issue(b, ci):
        pltpu.async_copy(gb[b], acc.at[row_v.at[ci]], ssem[b], add=True)

    def s_wait(b, ci):
        pltpu.make_async_copy(gb[b], acc.at[row_v.at[ci]], ssem[b]).wait()

    def scale(b, ci):
        def group_body(g, carry):
            vv = val_v[ci, pl.ds(g * 16, 16)]
            for e in range(16):
                v = vv[e]
                row = g * 16 + e
                for j in range(DH // 16):
                    sl = pl.ds(j * 16, 16)
                    gb[b][row, sl] = gb[b][row, sl] * v
            return carry

        lax.fori_loop(0, CH // 16, group_body, 0)

    def slot(ci, m, do_sswait=True, do_gather=True):
        X = m
        Z = (m + 2) % 3
        g_wait(X, ci)
        scale(X, ci)
        s_issue(X, ci)
        if do_sswait:
            s_wait(Z, ci - 1)
        if do_gather:
            g_issue(Z, ci + 2)

    def super_body(sc, carry0):
        pltpu.sync_copy(rows.at[s].at[sc], row_v)
        pltpu.sync_copy(cols.at[s].at[sc], col_v)
        pltpu.sync_copy(vals.at[s].at[sc], val_v)

        g_issue(0, 0)
        g_issue(1, 1)
        slot(0, 0, do_sswait=False)
        slot(1, 1)

        def block_body(u, carry):
            c0 = 2 + 3 * u
            for d in range(3):
                slot(c0 + d, (2 + d) % 3)
            return carry

        lax.fori_loop(0, (CPS - 5) // 3, block_body, 0)

        for ci in range(CPS - 3, CPS):
            slot(ci, ci % 3, do_gather=(ci + 2 < CPS))
        s_wait((CPS - 1) % 3, CPS - 1)
        return carry0

    lax.fori_loop(0, NSC, super_body, 0)
    plsc.subcore_barrier()

    osl = pl.ds(base, RPT)
    pltpu.sync_copy(acc.at[osl], out.at[c].at[osl])


_TM = 1000


def _linear_relu_body(hs_ref, wt_ref, o_ref):
    hl = hs_ref[0]
    hr = hs_ref[1]
    acc = jnp.dot(hl, wt_ref[:DH], preferred_element_type=jnp.float32)
    acc += jnp.dot(hr, wt_ref[DH:], preferred_element_type=jnp.float32)
    o_ref[...] = jnp.maximum(acc, 0.0)


_linear_relu = pl.pallas_call(
    _linear_relu_body,
    grid=(N // _TM,),
    in_specs=[
        pl.BlockSpec((NC, _TM, DH), lambda i: (0, i, 0)),
        pl.BlockSpec((D_IN, D_OUT), lambda i: (0, 0)),
    ],
    out_specs=pl.BlockSpec((_TM, D_OUT), lambda i: (i, 0)),
    out_shape=jax.ShapeDtypeStruct((N, D_OUT), jnp.float32),
)


def kernel(x, A_indices, A_values, shape, W):
    del shape
    pad = EPT - NNZ // NS
    spread = (jnp.arange(pad, dtype=jnp.int32)[None, :] * NS
              + jnp.arange(NS, dtype=jnp.int32)[:, None])
    rows = jnp.concatenate(
        [A_indices[0].astype(jnp.int32).reshape(NS, NNZ // NS), spread % N2],
        axis=1).reshape(NS, NSC, CPS, CH)
    cols = jnp.concatenate(
        [A_indices[1].astype(jnp.int32).reshape(NS, NNZ // NS), spread % N],
        axis=1).reshape(NS, NSC, CPS, CH)
    vals = jnp.pad(A_values.astype(jnp.float32).reshape(NS, NNZ // NS),
                   ((0, 0), (0, pad))).reshape(NS, NSC, CPS, CH)
    hs = _spmm_sc(x, rows, cols, vals)
    return _linear_relu(hs, W.T.astype(jnp.float32))
